# baseline (speedup 1.0000x reference)
import functools

import jax
import jax.numpy as jnp
from jax import lax
from jax.experimental import pallas as pl
from jax.experimental.pallas import tpu as pltpu
from jax.experimental.pallas import tpu_sc as plsc

D_MODEL = 768
NUM_WORKERS = 32
LANES = 16


def _tc_add_kernel(x_ref, pe_ref, o_ref):
    o_ref[...] = x_ref[...] + pe_ref[...]


def _tc_merge_kernel(buf_ref, s_ref, o_ref):
    del buf_ref
    o_ref[0] = s_ref[...]


def _sc_add_kernel(x_hbm, pe_hbm, o_hbm, xb, pb, ob,
                   sx0, sx1, sp0, sp1, so0, so1, *,
                   g0, sc_rows, pe_rows, chunk_rows):
    sx = (sx0, sx1)
    sp = (sp0, sp1)
    so = (so0, so1)
    w = lax.axis_index("s") * 2 + lax.axis_index("c")
    band = sc_rows // NUM_WORKERS
    n_chunks = band // chunk_rows
    groups = D_MODEL // LANES
    row_base = g0 + w * band
    pe_base = (g0 % pe_rows) + w * band
    out_base = w * band

    def in_copies(k, slot):
        r0 = row_base + k * chunk_rows
        pr0 = pe_base + k * chunk_rows
        return (
            pltpu.make_async_copy(
                pe_hbm.at[pl.ds(pr0, chunk_rows)], pb.at[slot], sp[slot]),
            pltpu.make_async_copy(
                x_hbm.at[pl.ds(r0, chunk_rows)], xb.at[slot], sx[slot]),
        )

    def out_copy(k, slot):
        r0 = out_base + k * chunk_rows
        return pltpu.make_async_copy(
            ob.at[slot], o_hbm.at[pl.ds(r0, chunk_rows)], so[slot])

    for c in in_copies(0, 0):
        c.start()
    for k in range(n_chunks):
        slot = k % 2
        other = 1 - slot
        if k + 1 < n_chunks:
            for c in in_copies(k + 1, other):
                c.start()
        if k >= 2:
            out_copy(k - 2, slot).wait()
        for c in in_copies(k, slot):
            c.wait()

        @plsc.parallel_loop(0, chunk_rows * groups, unroll=8)
        def _add(i):
            r = i // groups
            sl = pl.ds((i % groups) * LANES, LANES)
            ob[slot, r, sl] = xb[slot, r, sl] + pb[slot, r, sl]

        out_copy(k, slot).start()
    for k in (n_chunks - 2, n_chunks - 1):
        if k >= 0:
            out_copy(k, k % 2).wait()


def kernel(x, pe):
    B, S, D = x.shape
    rows = B * S
    pe_rows = pe.shape[0]

    S_BLK = 2048
    n_s = S // S_BLK
    sc_rows = 2 * S_BLK
    g0 = rows - sc_rows

    chunk_rows = 16
    mesh = plsc.VectorSubcoreMesh(core_axis_name="c", subcore_axis_name="s")
    sc = pl.kernel(
        functools.partial(_sc_add_kernel, g0=g0, sc_rows=sc_rows,
                          pe_rows=pe_rows, chunk_rows=chunk_rows),
        out_type=jax.ShapeDtypeStruct((sc_rows, D), jnp.float32),
        mesh=mesh,
        scratch_types=[
            pltpu.VMEM((2, chunk_rows, D), jnp.float32),
            pltpu.VMEM((2, chunk_rows, D), jnp.float32),
            pltpu.VMEM((2, chunk_rows, D), jnp.float32),
        ] + [pltpu.SemaphoreType.DMA] * 6,
    )
    sc_out = sc(x.reshape(rows, D), pe)

    sc_tiles = sc_rows // S_BLK
    full_tiles = n_s - sc_tiles
    head = full_tiles * B
    n_blocks = head + sc_tiles * (B - 1)

    def _bs(i):
        in_head = i < head
        return (jnp.where(in_head, i % B, (i - head) % (B - 1)),
                jnp.where(in_head, i // B, full_tiles + (i - head) // (B - 1)))

    def imap_x(i):
        b, s = _bs(i)
        return (b, s, 0)

    def imap_pe(i):
        _, s = _bs(i)
        return (s, 0)

    tc_out = pl.pallas_call(
        _tc_add_kernel,
        grid=(n_blocks,),
        in_specs=[
            pl.BlockSpec((1, S_BLK, D), imap_x),
            pl.BlockSpec((S_BLK, D), imap_pe),
        ],
        out_specs=pl.BlockSpec((1, S_BLK, D), imap_x),
        out_shape=jax.ShapeDtypeStruct((B, S, D), x.dtype),
    )(x, pe)

    return pl.pallas_call(
        _tc_merge_kernel,
        grid=(sc_tiles,),
        in_specs=[
            pl.BlockSpec(memory_space=pl.ANY),
            pl.BlockSpec((S_BLK, D), lambda i: (i, 0)),
        ],
        out_specs=pl.BlockSpec((1, S_BLK, D), lambda i: (B - 1, full_tiles + i, 0)),
        out_shape=jax.ShapeDtypeStruct((B, S, D), x.dtype),
        input_output_aliases={0: 0},
    )(tc_out, sc_out)


# speedup vs baseline: 1.0581x; 1.0581x over previous
import functools

import jax
import jax.numpy as jnp
from jax import lax
from jax.experimental import pallas as pl
from jax.experimental.pallas import tpu as pltpu
from jax.experimental.pallas import tpu_sc as plsc

D_MODEL = 768
NUM_WORKERS = 32
LANES = 16


def _tc_add_kernel(x_ref, pe_ref, o_ref):
    o_ref[...] = x_ref[...] + pe_ref[...]


def _tc_merge_kernel(buf_ref, s_ref, o_ref):
    del buf_ref
    o_ref[0] = s_ref[...]


def _sc_add_kernel(x_hbm, pe_hbm, o_hbm, xb, pb, ob,
                   sx0, sx1, sp0, sp1, so0, so1, *,
                   g0, sc_rows, pe_rows, chunk_rows):
    sx = (sx0, sx1)
    sp = (sp0, sp1)
    so = (so0, so1)
    w = lax.axis_index("s") * 2 + lax.axis_index("c")
    band = sc_rows // NUM_WORKERS
    n_chunks = band // chunk_rows
    groups = D_MODEL // LANES
    row_base = g0 + w * band
    pe_base = (g0 % pe_rows) + w * band
    out_base = w * band

    def in_copies(k, slot):
        r0 = row_base + k * chunk_rows
        pr0 = pe_base + k * chunk_rows
        return (
            pltpu.make_async_copy(
                pe_hbm.at[pl.ds(pr0, chunk_rows)], pb.at[slot], sp[slot]),
            pltpu.make_async_copy(
                x_hbm.at[pl.ds(r0, chunk_rows)], xb.at[slot], sx[slot]),
        )

    def out_copy(k, slot):
        r0 = out_base + k * chunk_rows
        return pltpu.make_async_copy(
            ob.at[slot], o_hbm.at[pl.ds(r0, chunk_rows)], so[slot])

    for c in in_copies(0, 0):
        c.start()
    for k in range(n_chunks):
        slot = k % 2
        other = 1 - slot
        if k + 1 < n_chunks:
            for c in in_copies(k + 1, other):
                c.start()
        if k >= 2:
            out_copy(k - 2, slot).wait()
        for c in in_copies(k, slot):
            c.wait()

        @plsc.parallel_loop(0, chunk_rows * groups, unroll=8)
        def _add(i):
            r = i // groups
            sl = pl.ds((i % groups) * LANES, LANES)
            ob[slot, r, sl] = xb[slot, r, sl] + pb[slot, r, sl]

        out_copy(k, slot).start()
    for k in (n_chunks - 2, n_chunks - 1):
        if k >= 0:
            out_copy(k, k % 2).wait()


def kernel(x, pe):
    B, S, D = x.shape
    rows = B * S
    pe_rows = pe.shape[0]

    S_BLK = 2048
    n_s = S // S_BLK
    sc_rows = S_BLK
    g0 = rows - sc_rows

    chunk_rows = 16
    mesh = plsc.VectorSubcoreMesh(core_axis_name="c", subcore_axis_name="s")
    sc = pl.kernel(
        functools.partial(_sc_add_kernel, g0=g0, sc_rows=sc_rows,
                          pe_rows=pe_rows, chunk_rows=chunk_rows),
        out_type=jax.ShapeDtypeStruct((sc_rows, D), jnp.float32),
        mesh=mesh,
        scratch_types=[
            pltpu.VMEM((2, chunk_rows, D), jnp.float32),
            pltpu.VMEM((2, chunk_rows, D), jnp.float32),
            pltpu.VMEM((2, chunk_rows, D), jnp.float32),
        ] + [pltpu.SemaphoreType.DMA] * 6,
    )
    sc_out = sc(x.reshape(rows, D), pe)

    sc_tiles = sc_rows // S_BLK
    full_tiles = n_s - sc_tiles
    head = full_tiles * B
    n_blocks = head + sc_tiles * (B - 1)

    def _bs(i):
        in_head = i < head
        return (jnp.where(in_head, i % B, (i - head) % (B - 1)),
                jnp.where(in_head, i // B, full_tiles + (i - head) // (B - 1)))

    def imap_x(i):
        b, s = _bs(i)
        return (b, s, 0)

    def imap_pe(i):
        _, s = _bs(i)
        return (s, 0)

    tc_out = pl.pallas_call(
        _tc_add_kernel,
        grid=(n_blocks,),
        in_specs=[
            pl.BlockSpec((1, S_BLK, D), imap_x),
            pl.BlockSpec((S_BLK, D), imap_pe),
        ],
        out_specs=pl.BlockSpec((1, S_BLK, D), imap_x),
        out_shape=jax.ShapeDtypeStruct((B, S, D), x.dtype),
    )(x, pe)

    return pl.pallas_call(
        _tc_merge_kernel,
        grid=(sc_tiles,),
        in_specs=[
            pl.BlockSpec(memory_space=pl.ANY),
            pl.BlockSpec((S_BLK, D), lambda i: (i, 0)),
        ],
        out_specs=pl.BlockSpec((1, S_BLK, D), lambda i: (B - 1, full_tiles + i, 0)),
        out_shape=jax.ShapeDtypeStruct((B, S, D), x.dtype),
        input_output_aliases={0: 0},
    )(tc_out, sc_out)
